# baseline (speedup 1.0000x reference)
import functools

import jax
import jax.numpy as jnp
from jax import lax
from jax.experimental import pallas as pl
from jax.experimental.pallas import tpu as pltpu
from jax.experimental.pallas import tpu_sc as plsc

N = 10000
NP = 10240
HP = 32
NC, NS = 2, 16
NW = NC * NS
CH = 128
NCH = 84
ECH = 168
RPT = NP // NS
ND = 10240
DPT = ND // NS

_sc_mesh = plsc.VectorSubcoreMesh(core_axis_name="c", subcore_axis_name="s")


def _rsqrt16(x):
    i = lax.bitcast_convert_type(x, jnp.int32)
    y = lax.bitcast_convert_type(
        jnp.int32(0x5F3759DF) - lax.shift_right_logical(i, 1), jnp.float32)
    for _ in range(4):
        y = y * (1.5 - 0.5 * x * y * y)
    return y


@functools.partial(
    pl.kernel,
    out_type=jax.ShapeDtypeStruct((ND,), jnp.float32),
    mesh=_sc_mesh,
    scratch_types=[
        pltpu.VMEM_SHARED((ND,), jnp.float32),
        pltpu.VMEM((DPT,), jnp.float32),
        pltpu.VMEM((ECH, CH), jnp.int32),
        pltpu.VMEM((ECH, CH), jnp.float32),
        pltpu.SemaphoreType.DMA,
    ],
    compiler_params=pltpu.CompilerParams(use_tc_tiling_on_sc=False),
)
def _sc_degree(dst3, ew3, dinv_out, acc, zbuf, dstv, ewv, ss):
    cid = lax.axis_index("c")
    sid = lax.axis_index("s")
    z = jnp.zeros((16,), jnp.float32)

    def zero_body(i, carry):
        zbuf[pl.ds(i * 16, 16)] = z
        return carry

    lax.fori_loop(0, DPT // 16, zero_body, 0)
    pltpu.sync_copy(zbuf, acc.at[pl.ds(sid * DPT, DPT)])
    pltpu.sync_copy(dst3.at[sid], dstv)
    pltpu.sync_copy(ew3.at[sid], ewv)
    plsc.subcore_barrier()

    def body(c, carry):
        pltpu.async_copy(ewv.at[c], acc.at[dstv.at[c]], ss, add=True)

        @pl.when(c >= 8)
        def _():
            pltpu.make_async_copy(ewv.at[0], acc.at[dstv.at[0]], ss).wait()

        return carry

    lax.fori_loop(0, ECH, body, 0)

    def drain(c, carry):
        pltpu.make_async_copy(ewv.at[0], acc.at[dstv.at[0]], ss).wait()
        return carry

    lax.fori_loop(0, 8, drain, 0)
    plsc.subcore_barrier()

    pltpu.sync_copy(acc.at[pl.ds(sid * DPT, DPT)], zbuf)

    def rs_body(g, carry):
        sl = pl.ds(g * 16, 16)
        zbuf[sl] = _rsqrt16(zbuf[sl])
        return carry

    lax.fori_loop(0, DPT // 16, rs_body, 0)

    @pl.when(cid == 0)
    def _():
        pltpu.sync_copy(zbuf, dinv_out.at[pl.ds(sid * DPT, DPT)])


@functools.partial(
    pl.kernel,
    out_type=jax.ShapeDtypeStruct((NC, NP, HP), jnp.float32),
    mesh=_sc_mesh,
    scratch_types=[
        pltpu.VMEM_SHARED((NP, HP), jnp.float32),
        pltpu.VMEM((RPT, HP), jnp.float32),
        pltpu.VMEM((NCH, CH), jnp.int32),
        pltpu.VMEM((NCH, CH), jnp.int32),
        pltpu.VMEM((NCH, CH), jnp.float32),
        pltpu.VMEM((4, CH), jnp.float32),
        pltpu.VMEM((4, CH, HP), jnp.float32),
        pltpu.VMEM((4, CH, HP), jnp.float32),
    ] + [pltpu.SemaphoreType.DMA] * 12,
    compiler_params=pltpu.CompilerParams(use_tc_tiling_on_sc=False),
)
def _sc_spmm(h_hbm, dinv_hbm, src3, dst3, ew3, out_hbm,
             acc, zbuf, srcv, dstv, ewv, dv, rg, rs, *sems):
    sgs, sds, sss = sems[0:4], sems[4:8], sems[8:12]
    cid = lax.axis_index("c")
    sid = lax.axis_index("s")
    wid = sid * NC + cid
    z = jnp.zeros((16,), jnp.float32)

    def zero_body(i, carry):
        zbuf[i, pl.ds(0, 16)] = z
        zbuf[i, pl.ds(16, 16)] = z
        return carry

    lax.fori_loop(0, RPT, zero_body, 0)
    pltpu.sync_copy(zbuf, acc.at[pl.ds(sid * RPT, RPT)])
    pltpu.sync_copy(src3.at[wid], srcv)
    pltpu.sync_copy(dst3.at[wid], dstv)
    pltpu.sync_copy(ew3.at[wid], ewv)
    plsc.subcore_barrier()

    for b in range(4):
        pltpu.async_copy(h_hbm.at[srcv.at[b]], rg.at[b], sgs[b])
        pltpu.async_copy(dinv_hbm.at[srcv.at[b]], dv.at[b], sds[b])

    def body(i, carry):
        for b in range(4):
            sg, sd, ss = sgs[b], sds[b], sss[b]
            c = i * 4 + b
            pltpu.make_async_copy(h_hbm.at[srcv.at[c]], rg.at[b], sg).wait()
            pltpu.make_async_copy(dinv_hbm.at[srcv.at[c]], dv.at[b], sd).wait()

            @pl.when(i > 0)
            def _():
                pltpu.make_async_copy(rs.at[b], acc.at[dstv.at[c]], ss).wait()

            def scale(gi, c2):
                sl = pl.ds(gi * 16, 16)
                wv = ewv[c, sl] * dv[b, sl]
                for j in range(16):
                    e = gi * 16 + j
                    w = wv[j]
                    rs[b, e, pl.ds(0, 16)] = rg[b, e, pl.ds(0, 16)] * w
                    rs[b, e, pl.ds(16, 16)] = rg[b, e, pl.ds(16, 16)] * w
                return c2

            lax.fori_loop(0, CH // 16, scale, 0)

            @pl.when(i < NCH // 4 - 1)
            def _():
                pltpu.async_copy(h_hbm.at[srcv.at[c + 4]], rg.at[b], sg)
                pltpu.async_copy(dinv_hbm.at[srcv.at[c + 4]], dv.at[b], sd)

            pltpu.async_copy(rs.at[b], acc.at[dstv.at[c]], ss, add=True)
        return carry

    lax.fori_loop(0, NCH // 4, body, 0)
    for b in range(4):
        pltpu.make_async_copy(rs.at[b], acc.at[dstv.at[0]], sss[b]).wait()
    plsc.subcore_barrier()
    pltpu.sync_copy(acc.at[pl.ds(sid * RPT, RPT)], zbuf)
    pltpu.sync_copy(zbuf, out_hbm.at[cid, pl.ds(sid * RPT, RPT)])


def _tc_pre_body(x, w1t, h1_o):
    h1_o[...] = jnp.dot(x[...], w1t[...], preferred_element_type=jnp.float32)


def _tc_mid_body(aggp, dinv_r, bp, wt, out_o, gn_o):
    n = dinv_r.shape[0]
    dinv = dinv_r[...]
    pre = (aggp[0, :n] + aggp[1, :n]) * dinv + bp[...]
    nrm = jnp.sqrt(jnp.sum(pre * pre, axis=1, keepdims=True))
    o = jnp.maximum(pre / jnp.maximum(nrm, 1e-12), 0.0)
    out_o[...] = o
    gn_o[...] = jnp.dot(o, wt[...], preferred_element_type=jnp.float32)


def _tc_fin_body(aggp, dinv_r, bp, out1, out2, lw1, lw2, lw3, lb, y_o):
    n = dinv_r.shape[0]
    dinv = dinv_r[...]
    pre = (aggp[0, :n] + aggp[1, :n]) * dinv + bp[...]
    nrm = jnp.sqrt(jnp.sum(pre * pre, axis=1, keepdims=True))
    o3 = jnp.maximum(pre / jnp.maximum(nrm, 1e-12), 0.0)
    y_o[...] = (
        jnp.dot(out1[...], lw1[...], preferred_element_type=jnp.float32)
        + jnp.dot(out2[...], lw2[...], preferred_element_type=jnp.float32)
        + jnp.dot(o3, lw3[...], preferred_element_type=jnp.float32)
        + lb[...]
    )


def kernel(x, edge_index, edge_weights, W1, b1, W2, b2, W3, b3, lin_W, lin_b):
    n, d_in = x.shape
    h = W1.shape[0]
    d_out = lin_W.shape[0]
    f32 = jnp.float32

    loop = jnp.arange(n, dtype=jnp.int32)
    src = jnp.concatenate([edge_index[0].astype(jnp.int32), loop])
    dst = jnp.concatenate([edge_index[1].astype(jnp.int32), loop])
    ew = jnp.concatenate([edge_weights.astype(f32), jnp.ones((n,), f32)])
    epad = NW * CH * NCH - ew.shape[0]
    src = jnp.pad(src, (0, epad))
    dst = jnp.pad(dst, (0, epad))
    ew = jnp.pad(ew, (0, epad))
    src3 = src.reshape(NW, NCH, CH)
    dst3 = dst.reshape(NW, NCH, CH)
    ew3 = ew.reshape(NW, NCH, CH)
    dst3d = dst.reshape(NS, ECH, CH)
    ew3d = ew.reshape(NS, ECH, CH)

    def padw_t(W, rows):
        return jnp.zeros((rows, HP), f32).at[:W.shape[1], :h].set(W.T.astype(f32))

    w1t = padw_t(W1, d_in)
    w2t = padw_t(W2, HP)
    w3t = padw_t(W3, HP)
    b1p = jnp.zeros((1, HP), f32).at[0, :h].set(b1.astype(f32))
    b2p = jnp.zeros((1, HP), f32).at[0, :h].set(b2.astype(f32))
    b3p = jnp.zeros((1, HP), f32).at[0, :h].set(b3.astype(f32))
    lws = [
        jnp.zeros((HP, d_out), f32).at[:h, :].set(lin_W[:, i * h:(i + 1) * h].T.astype(f32))
        for i in range(3)
    ]
    lbp = lin_b.astype(f32).reshape(1, d_out)

    dinv = _sc_degree(dst3d, ew3d)
    dinv_col = dinv[:n].reshape(n, 1)

    tc_pre = pl.pallas_call(
        _tc_pre_body,
        out_shape=jax.ShapeDtypeStruct((n, HP), f32),
    )
    tc_mid = pl.pallas_call(
        _tc_mid_body,
        out_shape=[jax.ShapeDtypeStruct((n, HP), f32),
                   jax.ShapeDtypeStruct((n, HP), f32)],
    )
    tc_fin = pl.pallas_call(
        _tc_fin_body,
        out_shape=jax.ShapeDtypeStruct((n, d_out), f32),
    )

    h1 = tc_pre(x.astype(f32), w1t)
    agg1 = _sc_spmm(h1, dinv, src3, dst3, ew3)
    out1, h2 = tc_mid(agg1, dinv_col, b1p, w2t)
    agg2 = _sc_spmm(h2, dinv, src3, dst3, ew3)
    out2, h3 = tc_mid(agg2, dinv_col, b2p, w3t)
    agg3 = _sc_spmm(h3, dinv, src3, dst3, ew3)
    return tc_fin(agg3, dinv_col, b3p, out1, out2, lws[0], lws[1], lws[2], lbp)


# speedup vs baseline: 1.2096x; 1.2096x over previous
import functools

import jax
import jax.numpy as jnp
from jax import lax
from jax.experimental import pallas as pl
from jax.experimental.pallas import tpu as pltpu
from jax.experimental.pallas import tpu_sc as plsc

N = 10000
NP = 10240
HP = 32
NC, NS = 2, 16
NW = NC * NS
CH = 128
NCH = 80
ECH = 160
RPT = NP // NS
ND = 10240
DPT = ND // NS

_sc_mesh = plsc.VectorSubcoreMesh(core_axis_name="c", subcore_axis_name="s")


def _rsqrt16(x):
    i = lax.bitcast_convert_type(x, jnp.int32)
    y = lax.bitcast_convert_type(
        jnp.int32(0x5F3759DF) - lax.shift_right_logical(i, 1), jnp.float32)
    for _ in range(4):
        y = y * (1.5 - 0.5 * x * y * y)
    return y


@functools.partial(
    pl.kernel,
    out_type=jax.ShapeDtypeStruct((ND,), jnp.float32),
    mesh=_sc_mesh,
    scratch_types=[
        pltpu.VMEM_SHARED((ND,), jnp.float32),
        pltpu.VMEM((DPT,), jnp.float32),
        pltpu.VMEM((ECH, CH), jnp.int32),
        pltpu.VMEM((ECH, CH), jnp.float32),
        pltpu.SemaphoreType.DMA,
    ],
    compiler_params=pltpu.CompilerParams(use_tc_tiling_on_sc=False),
)
def _sc_degree(dst3, ew3, dinv_out, acc, zbuf, dstv, ewv, ss):
    cid = lax.axis_index("c")
    sid = lax.axis_index("s")
    z = jnp.zeros((16,), jnp.float32)

    def zero_body(i, carry):
        zbuf[pl.ds(i * 16, 16)] = z
        return carry

    lax.fori_loop(0, DPT // 16, zero_body, 0)
    pltpu.sync_copy(zbuf, acc.at[pl.ds(sid * DPT, DPT)])
    pltpu.sync_copy(dst3.at[sid], dstv)
    pltpu.sync_copy(ew3.at[sid], ewv)
    plsc.subcore_barrier()

    def body(c, carry):
        pltpu.async_copy(ewv.at[c], acc.at[dstv.at[c]], ss, add=True)

        @pl.when(c >= 8)
        def _():
            pltpu.make_async_copy(ewv.at[0], acc.at[dstv.at[0]], ss).wait()

        return carry

    lax.fori_loop(0, ECH, body, 0)

    def drain(c, carry):
        pltpu.make_async_copy(ewv.at[0], acc.at[dstv.at[0]], ss).wait()
        return carry

    lax.fori_loop(0, 8, drain, 0)
    plsc.subcore_barrier()

    pltpu.sync_copy(acc.at[pl.ds(sid * DPT, DPT)], zbuf)

    def rs_body(g, carry):
        sl = pl.ds(g * 16, 16)
        zbuf[sl] = _rsqrt16(zbuf[sl] + 1.0)
        return carry

    lax.fori_loop(0, DPT // 16, rs_body, 0)

    @pl.when(cid == 0)
    def _():
        pltpu.sync_copy(zbuf, dinv_out.at[pl.ds(sid * DPT, DPT)])


@functools.partial(
    pl.kernel,
    out_type=jax.ShapeDtypeStruct((NC, NP, HP), jnp.float32),
    mesh=_sc_mesh,
    scratch_types=[
        pltpu.VMEM_SHARED((NP, HP), jnp.float32),
        pltpu.VMEM((RPT, HP), jnp.float32),
        pltpu.VMEM((NCH, CH), jnp.int32),
        pltpu.VMEM((NCH, CH), jnp.int32),
        pltpu.VMEM((NCH, CH), jnp.float32),
        pltpu.VMEM((4, CH), jnp.float32),
        pltpu.VMEM((4, CH, HP), jnp.float32),
        pltpu.VMEM((4, CH, HP), jnp.float32),
    ] + [pltpu.SemaphoreType.DMA] * 12,
    compiler_params=pltpu.CompilerParams(use_tc_tiling_on_sc=False),
)
def _sc_spmm(h_hbm, dinv_hbm, src3, dst3, ew3, out_hbm,
             acc, zbuf, srcv, dstv, ewv, dv, rg, rs, *sems):
    sgs, sds, sss = sems[0:4], sems[4:8], sems[8:12]
    cid = lax.axis_index("c")
    sid = lax.axis_index("s")
    wid = sid * NC + cid
    z = jnp.zeros((16,), jnp.float32)

    def zero_body(i, carry):
        zbuf[i, pl.ds(0, 16)] = z
        zbuf[i, pl.ds(16, 16)] = z
        return carry

    lax.fori_loop(0, RPT, zero_body, 0)
    pltpu.sync_copy(zbuf, acc.at[pl.ds(sid * RPT, RPT)])
    pltpu.sync_copy(src3.at[wid], srcv)
    pltpu.sync_copy(dst3.at[wid], dstv)
    pltpu.sync_copy(ew3.at[wid], ewv)
    plsc.subcore_barrier()

    for b in range(4):
        pltpu.async_copy(h_hbm.at[srcv.at[b]], rg.at[b], sgs[b])
        pltpu.async_copy(dinv_hbm.at[srcv.at[b]], dv.at[b], sds[b])

    def body(i, carry):
        for b in range(4):
            sg, sd, ss = sgs[b], sds[b], sss[b]
            c = i * 4 + b
            pltpu.make_async_copy(h_hbm.at[srcv.at[c]], rg.at[b], sg).wait()
            pltpu.make_async_copy(dinv_hbm.at[srcv.at[c]], dv.at[b], sd).wait()

            @pl.when(i > 0)
            def _():
                pltpu.make_async_copy(rs.at[b], acc.at[dstv.at[c]], ss).wait()

            def scale(gi, c2):
                sl = pl.ds(gi * 16, 16)
                wv = ewv[c, sl] * dv[b, sl]
                for j in range(16):
                    e = gi * 16 + j
                    w = wv[j]
                    rs[b, e, pl.ds(0, 16)] = rg[b, e, pl.ds(0, 16)] * w
                    rs[b, e, pl.ds(16, 16)] = rg[b, e, pl.ds(16, 16)] * w
                return c2

            lax.fori_loop(0, CH // 16, scale, 0)

            @pl.when(i < NCH // 4 - 1)
            def _():
                pltpu.async_copy(h_hbm.at[srcv.at[c + 4]], rg.at[b], sg)
                pltpu.async_copy(dinv_hbm.at[srcv.at[c + 4]], dv.at[b], sd)

            pltpu.async_copy(rs.at[b], acc.at[dstv.at[c]], ss, add=True)
        return carry

    lax.fori_loop(0, NCH // 4, body, 0)
    for b in range(4):
        pltpu.make_async_copy(rs.at[b], acc.at[dstv.at[0]], sss[b]).wait()
    plsc.subcore_barrier()
    pltpu.sync_copy(acc.at[pl.ds(sid * RPT, RPT)], zbuf)
    pltpu.sync_copy(zbuf, out_hbm.at[cid, pl.ds(sid * RPT, RPT)])


def _tc_pre_body(x, w1t, h1_o):
    h1_o[...] = jnp.dot(x[...], w1t[...], preferred_element_type=jnp.float32)


def _tc_mid_body(aggp, g, dinv_r, bp, wt, out_o, gn_o):
    n = g.shape[0]
    dinv = dinv_r[...]
    pre = (aggp[0, :n] + aggp[1, :n] + g[...] * dinv) * dinv + bp[...]
    nrm = jnp.sqrt(jnp.sum(pre * pre, axis=1, keepdims=True))
    o = jnp.maximum(pre / jnp.maximum(nrm, 1e-12), 0.0)
    out_o[...] = o
    gn_o[...] = jnp.dot(o, wt[...], preferred_element_type=jnp.float32)


def _tc_fin_body(aggp, g, dinv_r, bp, out1, out2, lw1, lw2, lw3, lb, y_o):
    n = g.shape[0]
    dinv = dinv_r[...]
    pre = (aggp[0, :n] + aggp[1, :n] + g[...] * dinv) * dinv + bp[...]
    nrm = jnp.sqrt(jnp.sum(pre * pre, axis=1, keepdims=True))
    o3 = jnp.maximum(pre / jnp.maximum(nrm, 1e-12), 0.0)
    y_o[...] = (
        jnp.dot(out1[...], lw1[...], preferred_element_type=jnp.float32)
        + jnp.dot(out2[...], lw2[...], preferred_element_type=jnp.float32)
        + jnp.dot(o3, lw3[...], preferred_element_type=jnp.float32)
        + lb[...]
    )


def kernel(x, edge_index, edge_weights, W1, b1, W2, b2, W3, b3, lin_W, lin_b):
    n, d_in = x.shape
    h = W1.shape[0]
    d_out = lin_W.shape[0]
    f32 = jnp.float32

    src = edge_index[0].astype(jnp.int32)
    dst = edge_index[1].astype(jnp.int32)
    ew = edge_weights.astype(f32)
    epad = NW * CH * NCH - edge_weights.shape[0]
    src = jnp.pad(src, (0, epad))
    dst = jnp.pad(dst, (0, epad))
    ew = jnp.pad(ew, (0, epad))
    src3 = src.reshape(NW, NCH, CH)
    dst3 = dst.reshape(NW, NCH, CH)
    ew3 = ew.reshape(NW, NCH, CH)
    dst3d = dst.reshape(NS, ECH, CH)
    ew3d = ew.reshape(NS, ECH, CH)

    def padw_t(W, rows):
        return jnp.zeros((rows, HP), f32).at[:W.shape[1], :h].set(W.T.astype(f32))

    w1t = padw_t(W1, d_in)
    w2t = padw_t(W2, HP)
    w3t = padw_t(W3, HP)
    b1p = jnp.zeros((1, HP), f32).at[0, :h].set(b1.astype(f32))
    b2p = jnp.zeros((1, HP), f32).at[0, :h].set(b2.astype(f32))
    b3p = jnp.zeros((1, HP), f32).at[0, :h].set(b3.astype(f32))
    lws = [
        jnp.zeros((HP, d_out), f32).at[:h, :].set(lin_W[:, i * h:(i + 1) * h].T.astype(f32))
        for i in range(3)
    ]
    lbp = lin_b.astype(f32).reshape(1, d_out)

    dinv = _sc_degree(dst3d, ew3d)
    dinv_col = dinv[:n].reshape(n, 1)

    tc_pre = pl.pallas_call(
        _tc_pre_body,
        out_shape=jax.ShapeDtypeStruct((n, HP), f32),
    )
    tc_mid = pl.pallas_call(
        _tc_mid_body,
        out_shape=[jax.ShapeDtypeStruct((n, HP), f32),
                   jax.ShapeDtypeStruct((n, HP), f32)],
    )
    tc_fin = pl.pallas_call(
        _tc_fin_body,
        out_shape=jax.ShapeDtypeStruct((n, d_out), f32),
    )

    h1 = tc_pre(x.astype(f32), w1t)
    agg1 = _sc_spmm(h1, dinv, src3, dst3, ew3)
    out1, h2 = tc_mid(agg1, h1, dinv_col, b1p, w2t)
    agg2 = _sc_spmm(h2, dinv, src3, dst3, ew3)
    out2, h3 = tc_mid(agg2, h2, dinv_col, b2p, w3t)
    agg3 = _sc_spmm(h3, dinv, src3, dst3, ew3)
    return tc_fin(agg3, h3, dinv_col, b3p, out1, out2, lws[0], lws[1], lws[2], lbp)
